# baseline (speedup 1.0000x reference)
import functools

import jax
import jax.numpy as jnp
from jax import lax
from jax.experimental import pallas as pl
from jax.experimental.pallas import tpu as pltpu
from jax.experimental.pallas import tpu_sc as plsc

N = 10000
E = 320000
D = 128

_NC = 2
_NS = 16
_TILES = _NC * _NS
_CH = 80
_NCHUNK = 125
_RPT = 624
_RPT_LAST = N - 15 * _RPT


_ROWS = 2000
_GRID = N // _ROWS

def _row_spec():
    return pl.BlockSpec((_ROWS, D), lambda i: (i, 0))

def _full_spec():
    return pl.BlockSpec((D, D), lambda i: (0, 0))

def _bias_spec():
    return pl.BlockSpec((1, D), lambda i: (0, 0))


def _enc_body(x, w0, b0, w1, b1, w2, b2, wg, o):
    h = jnp.maximum(jnp.dot(x[...], w0[...], preferred_element_type=jnp.float32) + b0[...], 0.0)
    h = jnp.maximum(jnp.dot(h, w1[...], preferred_element_type=jnp.float32) + b1[...], 0.0)
    h = jnp.dot(h, w2[...], preferred_element_type=jnp.float32) + b2[...]
    o[...] = jnp.dot(h, wg[...], preferred_element_type=jnp.float32)


def _encode_g0(x, w0t, b0, w1t, b1, w2t, b2, wg0t):
    return pl.pallas_call(
        _enc_body,
        grid=(_GRID,),
        in_specs=[_row_spec(), _full_spec(), _bias_spec(), _full_spec(),
                  _bias_spec(), _full_spec(), _bias_spec(), _full_spec()],
        out_specs=_row_spec(),
        out_shape=jax.ShapeDtypeStruct((N, D), jnp.float32),
    )(x, w0t, b0, w1t, b1, w2t, b2, wg0t)


def _comb_g1_body(p, bg, wg, o):
    h = jnp.maximum(p[0] + p[1] + bg[...], 0.0)
    o[...] = jnp.dot(h, wg[...], preferred_element_type=jnp.float32)


def _combine_g1(p, bg0, wg1t):
    return pl.pallas_call(
        _comb_g1_body,
        grid=(_GRID,),
        in_specs=[pl.BlockSpec((2, _ROWS, D), lambda i: (0, i, 0)),
                  _bias_spec(), _full_spec()],
        out_specs=_row_spec(),
        out_shape=jax.ShapeDtypeStruct((N, D), jnp.float32),
    )(p, bg0, wg1t)


def _dec_body(p, bg, w0, b0, w1, b1, w2, b2, o):
    h = jnp.maximum(p[0] + p[1] + bg[...], 0.0)
    h = jnp.maximum(jnp.dot(h, w0[...], preferred_element_type=jnp.float32) + b0[...], 0.0)
    h = jnp.maximum(jnp.dot(h, w1[...], preferred_element_type=jnp.float32) + b1[...], 0.0)
    o[...] = jnp.dot(h, w2[...], preferred_element_type=jnp.float32) + b2[...]


def _combine_decode(p, bg1, w0t, b0, w1t, b1, w2t, b2):
    return pl.pallas_call(
        _dec_body,
        grid=(_GRID,),
        in_specs=[pl.BlockSpec((2, _ROWS, D), lambda i: (0, i, 0)),
                  _bias_spec(), _full_spec(), _bias_spec(), _full_spec(),
                  _bias_spec(), _full_spec(), _bias_spec()],
        out_specs=_row_spec(),
        out_shape=jax.ShapeDtypeStruct((N, D), jnp.float32),
    )(p, bg1, w0t, b0, w1t, b1, w2t, b2)


def _edge_agg(m, src3, dst3, zrows_hbm):
    mesh = plsc.VectorSubcoreMesh(core_axis_name="c", subcore_axis_name="s")

    @functools.partial(
        pl.kernel,
        out_type=jax.ShapeDtypeStruct((_NC, N, D), jnp.float32),
        mesh=mesh,
        scratch_types=[
            pltpu.VMEM((4, 1, _CH), jnp.int32),
            pltpu.VMEM((4, 1, _CH), jnp.int32),
            pltpu.VMEM((4, _CH, D), jnp.float32),
            pltpu.VMEM_SHARED((N, D), jnp.float32),
            [pltpu.SemaphoreType.DMA] * 4,
            [pltpu.SemaphoreType.DMA] * 4,
            [pltpu.SemaphoreType.DMA] * 4,
        ],
    )
    def k(m_hbm, src_hbm, dst_hbm, z_hbm, out_hbm, src_v, dst_v, rows, acc,
          isems, gsems, ssems):
        c = lax.axis_index("c")
        s = lax.axis_index("s")
        wid = c * _NS + s

        @pl.when(s < _NS - 1)
        def _():
            pltpu.sync_copy(z_hbm.at[pl.ds(0, _RPT)],
                            acc.at[pl.ds(s * _RPT, _RPT)])

        @pl.when(s == _NS - 1)
        def _():
            pltpu.sync_copy(z_hbm, acc.at[pl.ds((_NS - 1) * _RPT, _RPT_LAST)])
        plsc.subcore_barrier()

        def idx_start(j, b):
            pltpu.async_copy(src_hbm.at[wid, pl.ds(j, 1)], src_v.at[b],
                             isems[b])
            pltpu.async_copy(dst_hbm.at[wid, pl.ds(j, 1)], dst_v.at[b],
                             isems[b])

        def idx_wait(j, b):
            pltpu.make_async_copy(src_hbm.at[wid, pl.ds(j, 1)], src_v.at[b],
                                  isems[b]).wait()
            pltpu.make_async_copy(dst_hbm.at[wid, pl.ds(j, 1)], dst_v.at[b],
                                  isems[b]).wait()

        def gather_start(b):
            pltpu.async_copy(m_hbm.at[src_v.at[b, 0]], rows.at[b], gsems[b])

        def gather_wait(b):
            pltpu.make_async_copy(m_hbm.at[src_v.at[b, 0]], rows.at[b],
                                  gsems[b]).wait()

        def scat_start(b):
            pltpu.async_copy(rows.at[b], acc.at[dst_v.at[b, 0]], ssems[b],
                             add=True)

        def scat_wait(b):
            pltpu.make_async_copy(rows.at[b], acc.at[dst_v.at[b, 0]],
                                  ssems[b]).wait()

        idx_start(0, 0)
        idx_start(1, 1)
        idx_wait(0, 0)
        gather_start(0)

        def outer(i):
            for b in range(4):
                j = i + b

                @pl.when(j < _NCHUNK)
                def _():
                    @pl.when(j >= 2)
                    def _():
                        scat_wait((b + 2) % 4)

                    @pl.when(j + 2 < _NCHUNK)
                    def _():
                        idx_start(j + 2, (b + 2) % 4)

                    @pl.when(j + 1 < _NCHUNK)
                    def _():
                        idx_wait(j + 1, (b + 1) % 4)
                        gather_start((b + 1) % 4)

                    gather_wait(b)
                    scat_start(b)
        pl.loop(0, _NCHUNK, step=4)(outer)
        scat_wait((_NCHUNK - 2) % 4)
        scat_wait((_NCHUNK - 1) % 4)

        plsc.subcore_barrier()

        @pl.when(s < _NS - 1)
        def _():
            pltpu.sync_copy(acc.at[pl.ds(s * _RPT, _RPT)],
                            out_hbm.at[c, pl.ds(s * _RPT, _RPT)])

        @pl.when(s == _NS - 1)
        def _():
            pltpu.sync_copy(acc.at[pl.ds((_NS - 1) * _RPT, _RPT_LAST)],
                            out_hbm.at[c, pl.ds((_NS - 1) * _RPT, _RPT_LAST)])

    return k(m, src3, dst3, zrows_hbm)


def kernel(x, edge_index, W_enc0, b_enc0, W_enc1, b_enc1, W_enc2, b_enc2,
           W_g0, b_g0, W_g1, b_g1, W_dec0, b_dec0, W_dec1, b_dec1,
           W_dec2, b_dec2):
    src3 = edge_index[0].reshape(_TILES, _NCHUNK, _CH)
    dst3 = edge_index[1].reshape(_TILES, _NCHUNK, _CH)
    zrows = jnp.zeros((_RPT_LAST, D), jnp.float32)

    r = lambda b: b.reshape(1, D)
    m1 = _encode_g0(x, W_enc0.T, r(b_enc0), W_enc1.T, r(b_enc1),
                    W_enc2.T, r(b_enc2), W_g0.T)
    p1 = _edge_agg(m1, src3, dst3, zrows)
    m2 = _combine_g1(p1, r(b_g0), W_g1.T)
    p2 = _edge_agg(m2, src3, dst3, zrows)
    out = _combine_decode(p2, r(b_g1), W_dec0.T, r(b_dec0), W_dec1.T,
                          r(b_dec1), W_dec2.T, r(b_dec2))
    return out


# speedup vs baseline: 1.0038x; 1.0038x over previous
import functools

import jax
import jax.numpy as jnp
from jax import lax
from jax.experimental import pallas as pl
from jax.experimental.pallas import tpu as pltpu
from jax.experimental.pallas import tpu_sc as plsc

N = 10000
E = 320000
D = 128

_NC = 2
_NS = 16
_TILES = _NC * _NS
_CH = 80
_NCHUNK = 125
_RPT = 624
_RPT_LAST = N - 15 * _RPT


_ROWS = 5000
_GRID = N // _ROWS

def _row_spec():
    return pl.BlockSpec((_ROWS, D), lambda i: (i, 0))

def _full_spec():
    return pl.BlockSpec((D, D), lambda i: (0, 0))

def _bias_spec():
    return pl.BlockSpec((1, D), lambda i: (0, 0))


def _enc_body(x, w0, b0, w1, b1, w2, b2, wg, o):
    h = jnp.maximum(jnp.dot(x[...], w0[...], preferred_element_type=jnp.float32) + b0[...], 0.0)
    h = jnp.maximum(jnp.dot(h, w1[...], preferred_element_type=jnp.float32) + b1[...], 0.0)
    h = jnp.dot(h, w2[...], preferred_element_type=jnp.float32) + b2[...]
    o[...] = jnp.dot(h, wg[...], preferred_element_type=jnp.float32)


def _encode_g0(x, w0t, b0, w1t, b1, w2t, b2, wg0t):
    return pl.pallas_call(
        _enc_body,
        grid=(_GRID,),
        in_specs=[_row_spec(), _full_spec(), _bias_spec(), _full_spec(),
                  _bias_spec(), _full_spec(), _bias_spec(), _full_spec()],
        out_specs=_row_spec(),
        out_shape=jax.ShapeDtypeStruct((N, D), jnp.float32),
    )(x, w0t, b0, w1t, b1, w2t, b2, wg0t)


def _comb_g1_body(p, bg, wg, o):
    h = jnp.maximum(p[0] + p[1] + bg[...], 0.0)
    o[...] = jnp.dot(h, wg[...], preferred_element_type=jnp.float32)


def _combine_g1(p, bg0, wg1t):
    return pl.pallas_call(
        _comb_g1_body,
        grid=(_GRID,),
        in_specs=[pl.BlockSpec((2, _ROWS, D), lambda i: (0, i, 0)),
                  _bias_spec(), _full_spec()],
        out_specs=_row_spec(),
        out_shape=jax.ShapeDtypeStruct((N, D), jnp.float32),
    )(p, bg0, wg1t)


def _dec_body(p, bg, w0, b0, w1, b1, w2, b2, o):
    h = jnp.maximum(p[0] + p[1] + bg[...], 0.0)
    h = jnp.maximum(jnp.dot(h, w0[...], preferred_element_type=jnp.float32) + b0[...], 0.0)
    h = jnp.maximum(jnp.dot(h, w1[...], preferred_element_type=jnp.float32) + b1[...], 0.0)
    o[...] = jnp.dot(h, w2[...], preferred_element_type=jnp.float32) + b2[...]


def _combine_decode(p, bg1, w0t, b0, w1t, b1, w2t, b2):
    return pl.pallas_call(
        _dec_body,
        grid=(_GRID,),
        in_specs=[pl.BlockSpec((2, _ROWS, D), lambda i: (0, i, 0)),
                  _bias_spec(), _full_spec(), _bias_spec(), _full_spec(),
                  _bias_spec(), _full_spec(), _bias_spec()],
        out_specs=_row_spec(),
        out_shape=jax.ShapeDtypeStruct((N, D), jnp.float32),
    )(p, bg1, w0t, b0, w1t, b1, w2t, b2)


def _edge_agg(m, src3, dst3, zrows_hbm):
    mesh = plsc.VectorSubcoreMesh(core_axis_name="c", subcore_axis_name="s")

    @functools.partial(
        pl.kernel,
        out_type=jax.ShapeDtypeStruct((_NC, N, D), jnp.float32),
        mesh=mesh,
        scratch_types=[
            pltpu.VMEM((4, 1, _CH), jnp.int32),
            pltpu.VMEM((4, 1, _CH), jnp.int32),
            pltpu.VMEM((4, _CH, D), jnp.float32),
            pltpu.VMEM_SHARED((N, D), jnp.float32),
            [pltpu.SemaphoreType.DMA] * 4,
            [pltpu.SemaphoreType.DMA] * 4,
            [pltpu.SemaphoreType.DMA] * 4,
        ],
    )
    def k(m_hbm, src_hbm, dst_hbm, z_hbm, out_hbm, src_v, dst_v, rows, acc,
          isems, gsems, ssems):
        c = lax.axis_index("c")
        s = lax.axis_index("s")
        wid = c * _NS + s

        @pl.when(s < _NS - 1)
        def _():
            pltpu.sync_copy(z_hbm.at[pl.ds(0, _RPT)],
                            acc.at[pl.ds(s * _RPT, _RPT)])

        @pl.when(s == _NS - 1)
        def _():
            pltpu.sync_copy(z_hbm, acc.at[pl.ds((_NS - 1) * _RPT, _RPT_LAST)])
        plsc.subcore_barrier()

        def idx_start(j, b):
            pltpu.async_copy(src_hbm.at[wid, pl.ds(j, 1)], src_v.at[b],
                             isems[b])
            pltpu.async_copy(dst_hbm.at[wid, pl.ds(j, 1)], dst_v.at[b],
                             isems[b])

        def idx_wait(j, b):
            pltpu.make_async_copy(src_hbm.at[wid, pl.ds(j, 1)], src_v.at[b],
                                  isems[b]).wait()
            pltpu.make_async_copy(dst_hbm.at[wid, pl.ds(j, 1)], dst_v.at[b],
                                  isems[b]).wait()

        def gather_start(b):
            pltpu.async_copy(m_hbm.at[src_v.at[b, 0]], rows.at[b], gsems[b])

        def gather_wait(b):
            pltpu.make_async_copy(m_hbm.at[src_v.at[b, 0]], rows.at[b],
                                  gsems[b]).wait()

        def scat_start(b):
            pltpu.async_copy(rows.at[b], acc.at[dst_v.at[b, 0]], ssems[b],
                             add=True)

        def scat_wait(b):
            pltpu.make_async_copy(rows.at[b], acc.at[dst_v.at[b, 0]],
                                  ssems[b]).wait()

        idx_start(0, 0)
        idx_start(1, 1)
        idx_wait(0, 0)
        gather_start(0)

        def outer(i):
            for b in range(4):
                j = i + b

                @pl.when(j < _NCHUNK)
                def _():
                    @pl.when(j >= 2)
                    def _():
                        scat_wait((b + 2) % 4)

                    @pl.when(j + 2 < _NCHUNK)
                    def _():
                        idx_start(j + 2, (b + 2) % 4)

                    @pl.when(j + 1 < _NCHUNK)
                    def _():
                        idx_wait(j + 1, (b + 1) % 4)
                        gather_start((b + 1) % 4)

                    gather_wait(b)
                    scat_start(b)
        pl.loop(0, _NCHUNK, step=4)(outer)
        scat_wait((_NCHUNK - 2) % 4)
        scat_wait((_NCHUNK - 1) % 4)

        plsc.subcore_barrier()

        @pl.when(s < _NS - 1)
        def _():
            pltpu.sync_copy(acc.at[pl.ds(s * _RPT, _RPT)],
                            out_hbm.at[c, pl.ds(s * _RPT, _RPT)])

        @pl.when(s == _NS - 1)
        def _():
            pltpu.sync_copy(acc.at[pl.ds((_NS - 1) * _RPT, _RPT_LAST)],
                            out_hbm.at[c, pl.ds((_NS - 1) * _RPT, _RPT_LAST)])

    return k(m, src3, dst3, zrows_hbm)


def kernel(x, edge_index, W_enc0, b_enc0, W_enc1, b_enc1, W_enc2, b_enc2,
           W_g0, b_g0, W_g1, b_g1, W_dec0, b_dec0, W_dec1, b_dec1,
           W_dec2, b_dec2):
    src3 = edge_index[0].reshape(_TILES, _NCHUNK, _CH)
    dst3 = edge_index[1].reshape(_TILES, _NCHUNK, _CH)
    zrows = jnp.zeros((_RPT_LAST, D), jnp.float32)

    r = lambda b: b.reshape(1, D)
    m1 = _encode_g0(x, W_enc0.T, r(b_enc0), W_enc1.T, r(b_enc1),
                    W_enc2.T, r(b_enc2), W_g0.T)
    p1 = _edge_agg(m1, src3, dst3, zrows)
    m2 = _combine_g1(p1, r(b_g0), W_g1.T)
    p2 = _edge_agg(m2, src3, dst3, zrows)
    out = _combine_decode(p2, r(b_g1), W_dec0.T, r(b_dec0), W_dec1.T,
                          r(b_dec1), W_dec2.T, r(b_dec2))
    return out


# speedup vs baseline: 1.1096x; 1.1054x over previous
import functools

import jax
import jax.numpy as jnp
from jax import lax
from jax.experimental import pallas as pl
from jax.experimental.pallas import tpu as pltpu
from jax.experimental.pallas import tpu_sc as plsc

N = 10000
E = 320000
D = 128

_NC = 2
_NS = 16
_TILES = _NC * _NS
_CH = 128
_NCHUNKS = E // _CH
_RING = 3
_IRING = 6
_RPT = 624
_RPT_LAST = N - 15 * _RPT


_ROWS = 5000
_GRID = N // _ROWS

_DNT = (((1,), (1,)), ((), ()))

def _mm(x, w):
    return lax.dot_general(x, w, _DNT, preferred_element_type=jnp.float32)

def _row_spec():
    return pl.BlockSpec((_ROWS, D), lambda i: (i, 0))

def _full_spec():
    return pl.BlockSpec((D, D), lambda i: (0, 0))

def _bias_spec():
    return pl.BlockSpec((1, D), lambda i: (0, 0))


def _enc_body(x, w0, b0, w1, b1, w2, b2, wg, o):
    h = jnp.maximum(_mm(x[...], w0[...]) + b0[...], 0.0)
    h = jnp.maximum(_mm(h, w1[...]) + b1[...], 0.0)
    h = _mm(h, w2[...]) + b2[...]
    o[...] = _mm(h, wg[...])


def _encode_g0(x, w0, b0, w1, b1, w2, b2, wg0):
    return pl.pallas_call(
        _enc_body,
        grid=(_GRID,),
        in_specs=[_row_spec(), _full_spec(), _bias_spec(), _full_spec(),
                  _bias_spec(), _full_spec(), _bias_spec(), _full_spec()],
        out_specs=_row_spec(),
        out_shape=jax.ShapeDtypeStruct((N, D), jnp.float32),
    )(x, w0, b0, w1, b1, w2, b2, wg0)


def _comb_g1_body(p, bg, wg, o):
    h = jnp.maximum(p[0] + p[1] + bg[...], 0.0)
    o[...] = _mm(h, wg[...])


def _combine_g1(p, bg0, wg1):
    return pl.pallas_call(
        _comb_g1_body,
        grid=(_GRID,),
        in_specs=[pl.BlockSpec((2, _ROWS, D), lambda i: (0, i, 0)),
                  _bias_spec(), _full_spec()],
        out_specs=_row_spec(),
        out_shape=jax.ShapeDtypeStruct((N, D), jnp.float32),
    )(p, bg0, wg1)


def _dec_body(p, bg, w0, b0, w1, b1, w2, b2, o):
    h = jnp.maximum(p[0] + p[1] + bg[...], 0.0)
    h = jnp.maximum(_mm(h, w0[...]) + b0[...], 0.0)
    h = jnp.maximum(_mm(h, w1[...]) + b1[...], 0.0)
    o[...] = _mm(h, w2[...]) + b2[...]


def _combine_decode(p, bg1, w0, b0, w1, b1, w2, b2):
    return pl.pallas_call(
        _dec_body,
        grid=(_GRID,),
        in_specs=[pl.BlockSpec((2, _ROWS, D), lambda i: (0, i, 0)),
                  _bias_spec(), _full_spec(), _bias_spec(), _full_spec(),
                  _bias_spec(), _full_spec(), _bias_spec()],
        out_specs=_row_spec(),
        out_shape=jax.ShapeDtypeStruct((N, D), jnp.float32),
    )(p, bg1, w0, b0, w1, b1, w2, b2)


def _edge_agg(m, edge_index, zrows_hbm):
    mesh = plsc.VectorSubcoreMesh(core_axis_name="c", subcore_axis_name="s")

    @functools.partial(
        pl.kernel,
        out_type=jax.ShapeDtypeStruct((_NC, N, D), jnp.float32),
        mesh=mesh,
        scratch_types=[
            pltpu.VMEM((_IRING, 1, _CH), jnp.int32),
            pltpu.VMEM((_IRING, 1, _CH), jnp.int32),
            pltpu.VMEM((_RING, _CH, D), jnp.float32),
            pltpu.VMEM_SHARED((N, D), jnp.float32),
            [pltpu.SemaphoreType.DMA] * _IRING,
            [pltpu.SemaphoreType.DMA] * _RING,
            [pltpu.SemaphoreType.DMA] * _RING,
        ],
    )
    def k(m_hbm, edge_hbm, z_hbm, out_hbm, src_v, dst_v, rows, acc,
          isems, gsems, ssems):
        c = lax.axis_index("c")
        s = lax.axis_index("s")
        wid = c * _NS + s
        nch = jnp.where(wid < 4, 79, 78)
        base = wid * 78 + jnp.minimum(wid, 4)

        @pl.when(s < _NS - 1)
        def _():
            pltpu.sync_copy(z_hbm.at[pl.ds(0, _RPT)],
                            acc.at[pl.ds(s * _RPT, _RPT)])

        @pl.when(s == _NS - 1)
        def _():
            pltpu.sync_copy(z_hbm, acc.at[pl.ds((_NS - 1) * _RPT, _RPT_LAST)])
        plsc.subcore_barrier()

        def off(j):
            return (base + j) * _CH

        def idx_start(j, ib):
            pltpu.async_copy(edge_hbm.at[pl.ds(0, 1), pl.ds(off(j), _CH)],
                             src_v.at[ib], isems[ib])
            pltpu.async_copy(edge_hbm.at[pl.ds(1, 1), pl.ds(off(j), _CH)],
                             dst_v.at[ib], isems[ib])

        def idx_wait(j, ib):
            pltpu.make_async_copy(
                edge_hbm.at[pl.ds(0, 1), pl.ds(off(j), _CH)],
                src_v.at[ib], isems[ib]).wait()
            pltpu.make_async_copy(
                edge_hbm.at[pl.ds(1, 1), pl.ds(off(j), _CH)],
                dst_v.at[ib], isems[ib]).wait()

        def gather_start(ib, rb):
            pltpu.async_copy(m_hbm.at[src_v.at[ib, 0]], rows.at[rb],
                             gsems[rb])

        def gather_wait(ib, rb):
            pltpu.make_async_copy(m_hbm.at[src_v.at[ib, 0]], rows.at[rb],
                                  gsems[rb]).wait()

        def scat_start(ib, rb):
            pltpu.async_copy(rows.at[rb], acc.at[dst_v.at[ib, 0]], ssems[rb],
                             add=True)

        def scat_wait(ib, rb):
            pltpu.make_async_copy(rows.at[rb], acc.at[dst_v.at[ib, 0]],
                                  ssems[rb]).wait()

        idx_start(0, 0)
        idx_start(1, 1)
        idx_wait(0, 0)
        gather_start(0, 0)

        def outer(i):
            for u in range(_IRING):
                j = i + u

                @pl.when(j < nch)
                def _():
                    @pl.when(j >= 2)
                    def _():
                        scat_wait((u + 4) % _IRING, (u + 1) % _RING)

                    @pl.when(j + 2 < nch)
                    def _():
                        idx_start(j + 2, (u + 2) % _IRING)

                    @pl.when(j + 1 < nch)
                    def _():
                        idx_wait(j + 1, (u + 1) % _IRING)
                        gather_start((u + 1) % _IRING, (u + 1) % _RING)

                    gather_wait(u % _IRING, u % _RING)
                    scat_start(u % _IRING, u % _RING)
        pl.loop(0, nch, step=_IRING)(outer)

        @pl.when(wid < 4)
        def _():
            scat_wait(5, 2)
            scat_wait(0, 0)

        @pl.when(wid >= 4)
        def _():
            scat_wait(4, 1)
            scat_wait(5, 2)

        plsc.subcore_barrier()

        @pl.when(s < _NS - 1)
        def _():
            pltpu.sync_copy(acc.at[pl.ds(s * _RPT, _RPT)],
                            out_hbm.at[c, pl.ds(s * _RPT, _RPT)])

        @pl.when(s == _NS - 1)
        def _():
            pltpu.sync_copy(acc.at[pl.ds((_NS - 1) * _RPT, _RPT_LAST)],
                            out_hbm.at[c, pl.ds((_NS - 1) * _RPT, _RPT_LAST)])

    return k(m, edge_index, zrows_hbm)


def kernel(x, edge_index, W_enc0, b_enc0, W_enc1, b_enc1, W_enc2, b_enc2,
           W_g0, b_g0, W_g1, b_g1, W_dec0, b_dec0, W_dec1, b_dec1,
           W_dec2, b_dec2):
    zrows = jnp.zeros((_RPT_LAST, D), jnp.float32)

    r = lambda b: b.reshape(1, D)
    m1 = _encode_g0(x, W_enc0, r(b_enc0), W_enc1, r(b_enc1),
                    W_enc2, r(b_enc2), W_g0)
    p1 = _edge_agg(m1, edge_index, zrows)
    m2 = _combine_g1(p1, r(b_g0), W_g1)
    p2 = _edge_agg(m2, edge_index, zrows)
    out = _combine_decode(p2, r(b_g1), W_dec0, r(b_dec0), W_dec1,
                          r(b_dec1), W_dec2, r(b_dec2))
    return out


# speedup vs baseline: 1.1328x; 1.0209x over previous
import functools

import jax
import jax.numpy as jnp
from jax import lax
from jax.experimental import pallas as pl
from jax.experimental.pallas import tpu as pltpu
from jax.experimental.pallas import tpu_sc as plsc

N = 10000
E = 320000
D = 128

_NC = 2
_NS = 16
_TILES = _NC * _NS
_CH = 128
_NCHUNKS = E // _CH
_RING = 3
_IRING = 6
_RPT = 624
_RPT_LAST = N - 15 * _RPT


_ROWS = 5000
_GRID = N // _ROWS

_DNT = (((1,), (1,)), ((), ()))

def _mm(x, w):
    return lax.dot_general(x, w, _DNT, preferred_element_type=jnp.float32)

def _row_spec():
    return pl.BlockSpec((_ROWS, D), lambda i: (i, 0))

def _full_spec():
    return pl.BlockSpec((D, D), lambda i: (0, 0))

def _bias_spec():
    return pl.BlockSpec((1, D), lambda i: (0, 0))


def _enc_body(x, w0, b0, w1, b1, w2, b2, wg, o):
    h = jnp.maximum(_mm(x[...], w0[...]) + b0[...], 0.0)
    h = jnp.maximum(_mm(h, w1[...]) + b1[...], 0.0)
    h = _mm(h, w2[...]) + b2[...]
    o[...] = _mm(h, wg[...])


def _encode_g0(x, w0, b0, w1, b1, w2, b2, wg0):
    return pl.pallas_call(
        _enc_body,
        grid=(_GRID,),
        in_specs=[_row_spec(), _full_spec(), _bias_spec(), _full_spec(),
                  _bias_spec(), _full_spec(), _bias_spec(), _full_spec()],
        out_specs=_row_spec(),
        out_shape=jax.ShapeDtypeStruct((N, D), jnp.float32),
    )(x, w0, b0, w1, b1, w2, b2, wg0)


def _comb_g1_body(p, bg, wg, o):
    h = jnp.maximum(p[0] + p[1] + bg[...], 0.0)
    o[...] = _mm(h, wg[...])


def _combine_g1(p, bg0, wg1):
    return pl.pallas_call(
        _comb_g1_body,
        grid=(_GRID,),
        in_specs=[pl.BlockSpec((2, _ROWS, D), lambda i: (0, i, 0)),
                  _bias_spec(), _full_spec()],
        out_specs=_row_spec(),
        out_shape=jax.ShapeDtypeStruct((N, D), jnp.float32),
    )(p, bg0, wg1)


def _dec_body(p, bg, w0, b0, w1, b1, w2, b2, o):
    h = jnp.maximum(p[0] + p[1] + bg[...], 0.0)
    h = jnp.maximum(_mm(h, w0[...]) + b0[...], 0.0)
    h = jnp.maximum(_mm(h, w1[...]) + b1[...], 0.0)
    o[...] = _mm(h, w2[...]) + b2[...]


def _combine_decode(p, bg1, w0, b0, w1, b1, w2, b2):
    return pl.pallas_call(
        _dec_body,
        grid=(_GRID,),
        in_specs=[pl.BlockSpec((2, _ROWS, D), lambda i: (0, i, 0)),
                  _bias_spec(), _full_spec(), _bias_spec(), _full_spec(),
                  _bias_spec(), _full_spec(), _bias_spec()],
        out_specs=_row_spec(),
        out_shape=jax.ShapeDtypeStruct((N, D), jnp.float32),
    )(p, bg1, w0, b0, w1, b1, w2, b2)


def _edge_agg(m, edge_index, zrows_hbm):
    mesh = plsc.VectorSubcoreMesh(core_axis_name="c", subcore_axis_name="s")

    @functools.partial(
        pl.kernel,
        out_type=jax.ShapeDtypeStruct((_NC, N, D), jnp.float32),
        mesh=mesh,
        scratch_types=[
            pltpu.VMEM((_IRING, 1, _CH), jnp.int32),
            pltpu.VMEM((_IRING, 1, _CH), jnp.int32),
            pltpu.VMEM((_RING, _CH, D), jnp.float32),
            pltpu.VMEM_SHARED((N, D), jnp.float32),
            [pltpu.SemaphoreType.DMA] * _IRING,
            [pltpu.SemaphoreType.DMA] * _RING,
            [pltpu.SemaphoreType.DMA] * _RING,
            pltpu.SemaphoreType.DMA,
        ],
    )
    def k(m_hbm, edge_hbm, z_hbm, out_hbm, src_v, dst_v, rows, acc,
          isems, gsems, ssems, zsem):
        c = lax.axis_index("c")
        s = lax.axis_index("s")
        wid = c * _NS + s
        nch = jnp.where(wid < 4, 79, 78)
        base = wid * 78 + jnp.minimum(wid, 4)

        @pl.when(s < _NS - 1)
        def _():
            pltpu.async_copy(z_hbm.at[pl.ds(0, _RPT)],
                             acc.at[pl.ds(s * _RPT, _RPT)], zsem)

        @pl.when(s == _NS - 1)
        def _():
            pltpu.async_copy(z_hbm, acc.at[pl.ds((_NS - 1) * _RPT, _RPT_LAST)],
                             zsem)

        def off(j):
            return (base + j) * _CH

        def idx_start(j, ib):
            pltpu.async_copy(edge_hbm.at[pl.ds(0, 1), pl.ds(off(j), _CH)],
                             src_v.at[ib], isems[ib])
            pltpu.async_copy(edge_hbm.at[pl.ds(1, 1), pl.ds(off(j), _CH)],
                             dst_v.at[ib], isems[ib])

        def idx_wait(j, ib):
            pltpu.make_async_copy(
                edge_hbm.at[pl.ds(0, 1), pl.ds(off(j), _CH)],
                src_v.at[ib], isems[ib]).wait()
            pltpu.make_async_copy(
                edge_hbm.at[pl.ds(1, 1), pl.ds(off(j), _CH)],
                dst_v.at[ib], isems[ib]).wait()

        def gather_start(ib, rb):
            pltpu.async_copy(m_hbm.at[src_v.at[ib, 0]], rows.at[rb],
                             gsems[rb])

        def gather_wait(ib, rb):
            pltpu.make_async_copy(m_hbm.at[src_v.at[ib, 0]], rows.at[rb],
                                  gsems[rb]).wait()

        def scat_start(ib, rb):
            pltpu.async_copy(rows.at[rb], acc.at[dst_v.at[ib, 0]], ssems[rb],
                             add=True)

        def scat_wait(ib, rb):
            pltpu.make_async_copy(rows.at[rb], acc.at[dst_v.at[ib, 0]],
                                  ssems[rb]).wait()

        idx_start(0, 0)
        idx_start(1, 1)
        idx_wait(0, 0)
        gather_start(0, 0)

        @pl.when(s < _NS - 1)
        def _():
            pltpu.make_async_copy(z_hbm.at[pl.ds(0, _RPT)],
                                  acc.at[pl.ds(s * _RPT, _RPT)], zsem).wait()

        @pl.when(s == _NS - 1)
        def _():
            pltpu.make_async_copy(
                z_hbm, acc.at[pl.ds((_NS - 1) * _RPT, _RPT_LAST)], zsem).wait()
        plsc.subcore_barrier()

        def outer(i):
            for u in range(_IRING):
                j = i + u

                @pl.when(j < nch)
                def _():
                    @pl.when(j >= 2)
                    def _():
                        scat_wait((u + 4) % _IRING, (u + 1) % _RING)

                    @pl.when(j + 2 < nch)
                    def _():
                        idx_start(j + 2, (u + 2) % _IRING)

                    @pl.when(j + 1 < nch)
                    def _():
                        idx_wait(j + 1, (u + 1) % _IRING)
                        gather_start((u + 1) % _IRING, (u + 1) % _RING)

                    gather_wait(u % _IRING, u % _RING)
                    scat_start(u % _IRING, u % _RING)
        pl.loop(0, nch, step=_IRING)(outer)

        @pl.when(wid < 4)
        def _():
            scat_wait(5, 2)
            scat_wait(0, 0)

        @pl.when(wid >= 4)
        def _():
            scat_wait(4, 1)
            scat_wait(5, 2)

        plsc.subcore_barrier()

        @pl.when(s < _NS - 1)
        def _():
            pltpu.sync_copy(acc.at[pl.ds(s * _RPT, _RPT)],
                            out_hbm.at[c, pl.ds(s * _RPT, _RPT)])

        @pl.when(s == _NS - 1)
        def _():
            pltpu.sync_copy(acc.at[pl.ds((_NS - 1) * _RPT, _RPT_LAST)],
                            out_hbm.at[c, pl.ds((_NS - 1) * _RPT, _RPT_LAST)])

    return k(m, edge_index, zrows_hbm)


def kernel(x, edge_index, W_enc0, b_enc0, W_enc1, b_enc1, W_enc2, b_enc2,
           W_g0, b_g0, W_g1, b_g1, W_dec0, b_dec0, W_dec1, b_dec1,
           W_dec2, b_dec2):
    zrows = jnp.zeros((_RPT_LAST, D), jnp.float32)

    r = lambda b: b.reshape(1, D)
    m1 = _encode_g0(x, W_enc0, r(b_enc0), W_enc1, r(b_enc1),
                    W_enc2, r(b_enc2), W_g0)
    p1 = _edge_agg(m1, edge_index, zrows)
    m2 = _combine_g1(p1, r(b_g0), W_g1)
    p2 = _edge_agg(m2, edge_index, zrows)
    out = _combine_decode(p2, r(b_g1), W_dec0, r(b_dec0), W_dec1,
                          r(b_dec1), W_dec2, r(b_dec2))
    return out
